# baseline (speedup 1.0000x reference)
import functools

import jax
import jax.numpy as jnp
from jax.experimental import pallas as pl
from jax.experimental.pallas import tpu as pltpu


def _fused_heads_kernel(x_ref, w_ref, b_ref, coords_ref, pres_ref):
    y = jnp.dot(x_ref[...], w_ref[...], preferred_element_type=jnp.float32)
    y = y + b_ref[...]
    coords_ref[...] = y[:, 0:4]
    pres_ref[...] = y[:, 4:5]


@functools.partial(jax.jit, static_argnames=("tile",))
def _run(x, w, b, tile):
    n = x.shape[0]
    grid = (n // tile,)
    coords, pres = pl.pallas_call(
        _fused_heads_kernel,
        grid=grid,
        in_specs=[
            pl.BlockSpec((tile, x.shape[1]), lambda i: (i, 0)),
            pl.BlockSpec(w.shape, lambda i: (0, 0)),
            pl.BlockSpec(b.shape, lambda i: (0, 0)),
        ],
        out_specs=[
            pl.BlockSpec((tile, 4), lambda i: (i, 0)),
            pl.BlockSpec((tile, 1), lambda i: (i, 0)),
        ],
        out_shape=[
            jax.ShapeDtypeStruct((n, 4), jnp.float32),
            jax.ShapeDtypeStruct((n, 1), jnp.float32),
        ],
        compiler_params=pltpu.CompilerParams(
            dimension_semantics=("parallel",),
        ),
    )(x, w, b)
    return coords, pres


def kernel(local_features, W_coords, b_coords, W_pres, b_pres):
    B, C, R, D = local_features.shape
    n = B * C * R
    x = local_features.reshape(n, D)
    w = jnp.concatenate(
        [W_coords, W_pres, jnp.zeros((3, D), jnp.float32)], axis=0
    ).T
    b = jnp.concatenate(
        [b_coords, b_pres, jnp.zeros((3,), jnp.float32)]
    ).reshape(1, 8)
    coords, pres = _run(x, w, b, 3000)
    return (
        coords.reshape(B, C, R, 4),
        pres.reshape(B, C, R, 1),
    )


# speedup vs baseline: 1.0501x; 1.0501x over previous
import functools

import jax
import jax.numpy as jnp
from jax import lax
from jax.experimental import pallas as pl
from jax.experimental.pallas import tpu as pltpu


def _fused_heads_kernel(x_ref, wc_ref, wp_ref, bc_ref, bp_ref,
                        coords_ref, pres_ref):
    x = x_ref[...]
    w = jnp.concatenate([wc_ref[...], wp_ref[...]], axis=0)
    y = lax.dot_general(
        x, w,
        dimension_numbers=(((1,), (1,)), ((), ())),
        preferred_element_type=jnp.float32,
    )
    coords_ref[...] = y[:, 0:4] + bc_ref[...]
    pres_ref[...] = y[:, 4:5] + bp_ref[...]


@functools.partial(jax.jit, static_argnames=("tile",))
def _run(x, wc, wp, bc, bp, tile):
    n, d = x.shape
    grid = (n // tile,)
    coords, pres = pl.pallas_call(
        _fused_heads_kernel,
        grid=grid,
        in_specs=[
            pl.BlockSpec((tile, d), lambda i: (i, 0)),
            pl.BlockSpec(wc.shape, lambda i: (0, 0)),
            pl.BlockSpec(wp.shape, lambda i: (0, 0)),
            pl.BlockSpec(bc.shape, lambda i: (0, 0)),
            pl.BlockSpec(bp.shape, lambda i: (0, 0)),
        ],
        out_specs=[
            pl.BlockSpec((tile, 4), lambda i: (i, 0)),
            pl.BlockSpec((tile, 1), lambda i: (i, 0)),
        ],
        out_shape=[
            jax.ShapeDtypeStruct((n, 4), jnp.float32),
            jax.ShapeDtypeStruct((n, 1), jnp.float32),
        ],
        compiler_params=pltpu.CompilerParams(
            dimension_semantics=("arbitrary",),
        ),
    )(x, wc, wp, bc, bp)
    return coords, pres


def kernel(local_features, W_coords, b_coords, W_pres, b_pres):
    B, C, R, D = local_features.shape
    n = B * C * R
    x = local_features.reshape(n, D)
    coords, pres = _run(
        x, W_coords, W_pres,
        b_coords.reshape(1, 4), b_pres.reshape(1, 1),
        8000,
    )
    return (
        coords.reshape(B, C, R, 4),
        pres.reshape(B, C, R, 1),
    )
